# Initial kernel scaffold; baseline (speedup 1.0000x reference)
import jax
import jax.numpy as jnp
from jax.experimental import pallas as pl


def kernel(node_features, hierarchy_edges, hierarchy_levels, level_weights, level_biases):
    raise NotImplementedError("write your pallas kernel here")


# speedup vs baseline: 22.5053x; 22.5053x over previous
import functools

import jax
import jax.numpy as jnp
from jax import lax
from jax.experimental import pallas as pl
from jax.experimental.pallas import tpu as pltpu
from jax.experimental.pallas import tpu_sc as plsc

N = 10000
F = 128
E = 320000
LEVELS = 4

NC = 2
NS = 16
NW = NC * NS

C = 128
PER_TILE = -(-E // (NW * C)) * C
E_PAD = PER_TILE * NW
ACC_N = 10240
ROWS_PER_TILE = ACC_N // NS

NB = 5
BLK = N // NB


def _transform_body(x_ref, lv_ref, w_ref, b_ref, y_ref):
    x = x_ref[...]
    lv = lv_ref[0, 0, :]
    acc = jnp.zeros_like(x)
    for l in range(LEVELS):
        m = (lv == l).astype(jnp.float32)[:, None]
        xw = lax.dot_general(x * m, w_ref[l], (((1,), (1,)), ((), ())),
                             preferred_element_type=jnp.float32)
        acc = acc + xw + m * b_ref[l][None, :]
    y_ref[...] = 0.2 * acc


def _merge_body(x_ref, a_ref, o_ref):
    o_ref[...] = x_ref[...] + a_ref[0] + a_ref[1]


def _sc_body(y_hbm, src_hbm, dst_hbm, lv_hbm, zin_hbm, out_hbm,
             lv_v, src_v, dst_v, gidx_v, sidx_v, rows_v, acc_sh, sem):
    c = lax.axis_index("c")
    s = lax.axis_index("s")
    wid = s * NC + c

    pltpu.sync_copy(zin_hbm, acc_sh.at[pl.ds(s * ROWS_PER_TILE, ROWS_PER_TILE)])
    pltpu.sync_copy(lv_hbm, lv_v)
    pltpu.sync_copy(src_hbm.at[pl.ds(wid * PER_TILE, PER_TILE)], src_v)
    pltpu.sync_copy(dst_hbm.at[pl.ds(wid * PER_TILE, PER_TILE)], dst_v)
    plsc.subcore_barrier()

    def chunk(j, carry):
        base = j * C
        for v in range(C // 16):
            off = base + v * 16
            srcs = src_v[pl.ds(off, 16)]
            dsts = dst_v[pl.ds(off, 16)]
            ll = plsc.load_gather(lv_v, [srcs])
            hl = plsc.load_gather(lv_v, [dsts])
            valid = (ll >= 0) & (ll < LEVELS) & (hl > ll)
            dummy = N + v * 16 + lax.iota(jnp.int32, 16)
            gidx_v[pl.ds(v * 16, 16)] = srcs
            sidx_v[pl.ds(v * 16, 16)] = jnp.where(valid, dsts, dummy)
        pltpu.async_copy(y_hbm.at[gidx_v], rows_v, sem).wait()
        pltpu.sync_copy(rows_v, acc_sh.at[sidx_v], add=True)
        return carry

    lax.fori_loop(0, PER_TILE // C, chunk, 0)
    plsc.subcore_barrier()

    pltpu.sync_copy(acc_sh.at[pl.ds(s * ROWS_PER_TILE, ROWS_PER_TILE)],
                    out_hbm.at[c, pl.ds(s * ROWS_PER_TILE, ROWS_PER_TILE)])


_sc_edges = functools.partial(
    pl.kernel,
    out_type=jax.ShapeDtypeStruct((NC, ACC_N, F), jnp.float32),
    mesh=plsc.VectorSubcoreMesh(core_axis_name="c", subcore_axis_name="s"),
    compiler_params=pltpu.CompilerParams(needs_layout_passes=False),
    scratch_types=[
        pltpu.VMEM((N,), jnp.int32),
        pltpu.VMEM((PER_TILE,), jnp.int32),
        pltpu.VMEM((PER_TILE,), jnp.int32),
        pltpu.VMEM((C,), jnp.int32),
        pltpu.VMEM((C,), jnp.int32),
        pltpu.VMEM((C, F), jnp.float32),
        pltpu.VMEM_SHARED((ACC_N, F), jnp.float32),
        pltpu.SemaphoreType.DMA,
    ],
)(_sc_body)


def kernel(node_features, hierarchy_edges, hierarchy_levels, level_weights, level_biases):
    pad = E_PAD - E
    src_p = jnp.concatenate([hierarchy_edges[:, 0],
                             jnp.zeros((pad,), jnp.int32)])
    dst_p = jnp.concatenate([hierarchy_edges[:, 1],
                             jnp.zeros((pad,), jnp.int32)])
    lv3 = hierarchy_levels.reshape(NB, 1, BLK)

    y = pl.pallas_call(
        _transform_body,
        grid=(NB,),
        in_specs=[
            pl.BlockSpec((BLK, F), lambda i: (i, 0)),
            pl.BlockSpec((1, 1, BLK), lambda i: (i, 0, 0)),
            pl.BlockSpec((LEVELS, F, F), lambda i: (0, 0, 0)),
            pl.BlockSpec((LEVELS, F), lambda i: (0, 0)),
        ],
        out_specs=pl.BlockSpec((BLK, F), lambda i: (i, 0)),
        out_shape=jax.ShapeDtypeStruct((N, F), jnp.float32),
    )(node_features, lv3, level_weights, level_biases)

    zin = jnp.zeros((ROWS_PER_TILE, F), jnp.float32)
    parts = _sc_edges(y, src_p, dst_p, hierarchy_levels, zin)

    out = pl.pallas_call(
        _merge_body,
        grid=(NB,),
        in_specs=[
            pl.BlockSpec((BLK, F), lambda i: (i, 0)),
            pl.BlockSpec((NC, BLK, F), lambda i: (0, i, 0)),
        ],
        out_specs=pl.BlockSpec((BLK, F), lambda i: (i, 0)),
        out_shape=jax.ShapeDtypeStruct((N, F), jnp.float32),
    )(node_features, parts)
    return out


# speedup vs baseline: 54.7768x; 2.4340x over previous
import functools

import jax
import jax.numpy as jnp
from jax import lax
from jax.experimental import pallas as pl
from jax.experimental.pallas import tpu as pltpu
from jax.experimental.pallas import tpu_sc as plsc

N = 10000
F = 128
E = 320000
LEVELS = 4

NC = 2
NS = 16
NW = NC * NS

C = 64
PER_TILE = -(-E // (NW * 2 * C)) * 2 * C
E_PAD = PER_TILE * NW
ACC_N = 10240
ROWS_PER_TILE = ACC_N // NS
NVEC = PER_TILE // 16

NB = 5
BLK = N // NB


def _transform_body(x_ref, lv_ref, w_ref, b_ref, y_ref):
    x = x_ref[...]
    lv = lv_ref[0, 0, :]
    acc = jnp.zeros_like(x)
    for l in range(LEVELS):
        m = (lv == l).astype(jnp.float32)[:, None]
        xw = lax.dot_general(x * m, w_ref[l], (((1,), (1,)), ((), ())),
                             preferred_element_type=jnp.float32)
        acc = acc + xw + m * b_ref[l][None, :]
    y_ref[...] = 0.2 * acc


def _merge_body(x_ref, a_ref, o_ref):
    o_ref[...] = x_ref[...] + a_ref[0] + a_ref[1]


def _sc_body(y_hbm, src_hbm, dst_hbm, lv_hbm, zin_hbm, out_hbm,
             lv_v, gsrc_v, gdst_v,
             sidx_a, sidx_b, rows_a, rows_b, acc_sh, sem_a, sem_b):
    c = lax.axis_index("c")
    s = lax.axis_index("s")
    wid = s * NC + c

    pltpu.sync_copy(zin_hbm, acc_sh.at[pl.ds(s * ROWS_PER_TILE, ROWS_PER_TILE)])
    pltpu.sync_copy(lv_hbm, lv_v)
    pltpu.sync_copy(src_hbm.at[pl.ds(wid * PER_TILE, PER_TILE)],
                    gsrc_v.at[pl.ds(0, PER_TILE)])
    pltpu.sync_copy(dst_hbm.at[pl.ds(wid * PER_TILE, PER_TILE)],
                    gdst_v.at[pl.ds(0, PER_TILE)])
    plsc.subcore_barrier()

    def cvec(v, cnt):
        srcs = gsrc_v[pl.ds(v * 16, 16)]
        dsts = gdst_v[pl.ds(v * 16, 16)]
        ll = plsc.load_gather(lv_v, [srcs])
        hl = plsc.load_gather(lv_v, [dsts])
        valid = (ll >= 0) & (ll < LEVELS) & (hl > ll)
        plsc.store_compressed(gsrc_v.at[pl.ds(cnt, 16)], srcs, mask=valid)
        plsc.store_compressed(gdst_v.at[pl.ds(cnt, 16)], dsts, mask=valid)
        return cnt + plsc.all_reduce_population_count(valid)[0]

    cnt = lax.fori_loop(0, NVEC, cvec, jnp.int32(0))

    for v in range(C // 16):
        dummy = N + v * 16 + lax.iota(jnp.int32, 16)
        gsrc_v[pl.ds(cnt + v * 16, 16)] = jnp.zeros((16,), jnp.int32)
        gdst_v[pl.ds(cnt + v * 16, 16)] = dummy

    nch = (cnt + C - 1) // C

    def fill_sidx(j, sidx):
        for v in range(C // 16):
            sidx[pl.ds(v * 16, 16)] = gdst_v[pl.ds(j * C + v * 16, 16)]

    def start_gather(j, rows, sem):
        return pltpu.async_copy(y_hbm.at[gsrc_v.at[pl.ds(j * C, C)]], rows, sem)

    @pl.when(nch > 0)
    def _prologue():
        fill_sidx(0, sidx_a)
        start_gather(0, rows_a, sem_a)

    def pair(p, carry):
        j0 = 2 * p
        j1 = j0 + 1

        @pl.when(j1 < nch)
        def _startb():
            fill_sidx(j1, sidx_b)
            start_gather(j1, rows_b, sem_b)

        pltpu.make_async_copy(y_hbm.at[gsrc_v.at[pl.ds(0, C)]], rows_a, sem_a).wait()
        pltpu.sync_copy(rows_a, acc_sh.at[sidx_a], add=True)

        @pl.when(j0 + 2 < nch)
        def _starta():
            fill_sidx(j0 + 2, sidx_a)
            start_gather(j0 + 2, rows_a, sem_a)

        @pl.when(j1 < nch)
        def _drainb():
            pltpu.make_async_copy(y_hbm.at[gsrc_v.at[pl.ds(0, C)]], rows_b, sem_b).wait()
            pltpu.sync_copy(rows_b, acc_sh.at[sidx_b], add=True)

        return carry

    lax.fori_loop(0, (nch + 1) // 2, pair, jnp.int32(0))
    plsc.subcore_barrier()

    pltpu.sync_copy(acc_sh.at[pl.ds(s * ROWS_PER_TILE, ROWS_PER_TILE)],
                    out_hbm.at[c, pl.ds(s * ROWS_PER_TILE, ROWS_PER_TILE)])


_sc_edges = functools.partial(
    pl.kernel,
    out_type=jax.ShapeDtypeStruct((NC, ACC_N, F), jnp.float32),
    mesh=plsc.VectorSubcoreMesh(core_axis_name="c", subcore_axis_name="s"),
    compiler_params=pltpu.CompilerParams(needs_layout_passes=False),
    scratch_types=[
        pltpu.VMEM((N,), jnp.int32),
        pltpu.VMEM((PER_TILE + C,), jnp.int32),
        pltpu.VMEM((PER_TILE + C,), jnp.int32),
        pltpu.VMEM((C,), jnp.int32),
        pltpu.VMEM((C,), jnp.int32),
        pltpu.VMEM((C, F), jnp.float32),
        pltpu.VMEM((C, F), jnp.float32),
        pltpu.VMEM_SHARED((ACC_N, F), jnp.float32),
        pltpu.SemaphoreType.DMA,
        pltpu.SemaphoreType.DMA,
    ],
)(_sc_body)


def kernel(node_features, hierarchy_edges, hierarchy_levels, level_weights, level_biases):
    pad = E_PAD - E
    src_p = jnp.concatenate([hierarchy_edges[:, 0],
                             jnp.zeros((pad,), jnp.int32)])
    dst_p = jnp.concatenate([hierarchy_edges[:, 1],
                             jnp.zeros((pad,), jnp.int32)])
    lv3 = hierarchy_levels.reshape(NB, 1, BLK)

    y = pl.pallas_call(
        _transform_body,
        grid=(NB,),
        in_specs=[
            pl.BlockSpec((BLK, F), lambda i: (i, 0)),
            pl.BlockSpec((1, 1, BLK), lambda i: (i, 0, 0)),
            pl.BlockSpec((LEVELS, F, F), lambda i: (0, 0, 0)),
            pl.BlockSpec((LEVELS, F), lambda i: (0, 0)),
        ],
        out_specs=pl.BlockSpec((BLK, F), lambda i: (i, 0)),
        out_shape=jax.ShapeDtypeStruct((N, F), jnp.float32),
    )(node_features, lv3, level_weights, level_biases)

    zin = jnp.zeros((ROWS_PER_TILE, F), jnp.float32)
    parts = _sc_edges(y, src_p, dst_p, hierarchy_levels, zin)

    out = pl.pallas_call(
        _merge_body,
        grid=(NB,),
        in_specs=[
            pl.BlockSpec((BLK, F), lambda i: (i, 0)),
            pl.BlockSpec((NC, BLK, F), lambda i: (0, i, 0)),
        ],
        out_specs=pl.BlockSpec((BLK, F), lambda i: (i, 0)),
        out_shape=jax.ShapeDtypeStruct((N, F), jnp.float32),
    )(node_features, parts)
    return out
